# Initial kernel scaffold; baseline (speedup 1.0000x reference)
import jax
import jax.numpy as jnp
from jax.experimental import pallas as pl


def kernel(b, weights):
    raise NotImplementedError("write your pallas kernel here")


# speedup vs baseline: 2.4077x; 2.4077x over previous
import jax
import jax.numpy as jnp
from jax import lax
from jax.experimental import pallas as pl
from jax.experimental.pallas import tpu as pltpu
from jax.experimental.pallas import tpu_sc as plsc

M = 8
NSYM = 2 ** M
BATCH = 262144

NC, NS, L = 2, 16, 16
NW = NC * NS
CHUNK = BATCH // NW
BLOCKS = CHUNK // 128
BWORDS = 128 * M


def _rsqrt16(x):
    i = plsc.bitcast(x, jnp.int32)
    y = plsc.bitcast(jnp.int32(0x5F3759DF) - (i >> 1), jnp.float32)
    half = x * 0.5
    for _ in range(3):
        y = y * (1.5 - half * y * y)
    return y


def _allsum16(v, scratch):
    iota = lax.iota(jnp.int32, L)
    for k in (8, 4, 2, 1):
        scratch[pl.ds(0, L)] = v
        v = v + plsc.load_gather(scratch, [iota ^ k])
    return v


def _sc_body(bits_hbm, w_hbm, out_hbm, bits_v, tbl_v, out_v, red_v):
    wid = lax.axis_index("s") * NC + lax.axis_index("c")

    pltpu.sync_copy(bits_hbm.at[pl.ds(wid * (CHUNK * M), CHUNK * M)], bits_v)
    pltpu.sync_copy(w_hbm, tbl_v)

    acc = jnp.zeros((L,), jnp.float32)
    for i in range(2 * NSYM // L):
        v = tbl_v[pl.ds(i * L, L)]
        acc = acc + v * v
    inv = _rsqrt16(_allsum16(acc, red_v) * (1.0 / NSYM))
    for i in range(2 * NSYM // L):
        tbl_v[pl.ds(i * L, L)] = tbl_v[pl.ds(i * L, L)] * inv

    def body(t, _):
        bin_ = t * BWORDS
        bout = t * 256
        for gb in range(8):
            r0 = gb * L
            idx = bits_v[pl.ds(bin_ + r0, L)]
            for j in range(1, M):
                idx = (idx << 1) | bits_v[pl.ds(bin_ + j * 128 + r0, L)]
            pos = idx + ((idx >> 7) << 7)
            out_v[pl.ds(bout + r0, L)] = plsc.load_gather(tbl_v, [pos])
            out_v[pl.ds(bout + 128 + r0, L)] = plsc.load_gather(tbl_v, [pos + 128])
        return 0

    lax.fori_loop(0, BLOCKS, body, 0)

    pltpu.sync_copy(out_v, out_hbm.at[pl.ds(wid * (CHUNK * 2), CHUNK * 2)])


@jax.jit
def _mapper(bits_blocked, w_blocked):
    mesh = plsc.VectorSubcoreMesh(core_axis_name="c", subcore_axis_name="s")
    fn = pl.kernel(
        _sc_body,
        out_type=jax.ShapeDtypeStruct((2 * BATCH,), jnp.float32),
        mesh=mesh,
        scratch_types=[
            pltpu.VMEM((CHUNK * M,), jnp.int32),
            pltpu.VMEM((2 * NSYM,), jnp.float32),
            pltpu.VMEM((CHUNK * 2,), jnp.float32),
            pltpu.VMEM((128,), jnp.float32),
        ],
        compiler_params=pltpu.CompilerParams(needs_layout_passes=False),
    )
    return fn(bits_blocked, w_blocked)


def kernel(b, weights):
    bits = b.reshape(BATCH // 128, 128, M).transpose(0, 2, 1).reshape(-1)
    w = weights.reshape(2, 128, 2).transpose(0, 2, 1).reshape(-1)
    out = _mapper(bits, w)
    return (
        out.reshape(BATCH // 128, 2, 128)
        .transpose(0, 2, 1)
        .reshape(BATCH, 1, 2)
    )


# speedup vs baseline: 2.6067x; 1.0827x over previous
import jax
import jax.numpy as jnp
from jax import lax
from jax.experimental import pallas as pl
from jax.experimental.pallas import tpu as pltpu
from jax.experimental.pallas import tpu_sc as plsc

M = 8
NSYM = 2 ** M
BATCH = 262144

NC, NS, L = 2, 16, 16
NW = NC * NS
CHUNK = BATCH // NW
BLOCKS = CHUNK // 128
BWORDS = 128 * M


def _rsqrt16(x):
    i = plsc.bitcast(x, jnp.int32)
    y = plsc.bitcast(jnp.int32(0x5F3759DF) - (i >> 1), jnp.float32)
    half = x * 0.5
    for _ in range(3):
        y = y * (1.5 - half * y * y)
    return y


def _allsum16(v, scratch):
    iota = lax.iota(jnp.int32, L)
    for k in (8, 4, 2, 1):
        scratch[pl.ds(0, L)] = v
        v = v + plsc.load_gather(scratch, [iota ^ k])
    return v


NSUB = 4
SBLK = BLOCKS // NSUB


def _sc_body(bits_hbm, w_hbm, out_hbm, bits_v, tbl_v, out_v, red_v, *sems):
    wid = lax.axis_index("s") * NC + lax.axis_index("c")
    ibase = wid * (CHUNK * M)

    copies = [
        pltpu.async_copy(
            bits_hbm.at[pl.ds(ibase + s * SBLK * BWORDS, SBLK * BWORDS)],
            bits_v.at[pl.ds(s * SBLK * BWORDS, SBLK * BWORDS)],
            sems[s],
        )
        for s in range(NSUB)
    ]
    pltpu.sync_copy(w_hbm, tbl_v)

    acc = jnp.zeros((L,), jnp.float32)
    for i in range(2 * NSYM // L):
        v = tbl_v[pl.ds(i * L, L)]
        acc = acc + v * v
    inv = _rsqrt16(_allsum16(acc, red_v) * (1.0 / NSYM))
    for i in range(2 * NSYM // L):
        tbl_v[pl.ds(i * L, L)] = tbl_v[pl.ds(i * L, L)] * inv

    for s in range(NSUB):
        copies[s].wait()

        @plsc.parallel_loop(s * SBLK, (s + 1) * SBLK)
        def _(t):
            bin_ = t * BWORDS
            bout = t * 256
            for gb in range(8):
                r0 = gb * L
                b = [bits_v[pl.ds(bin_ + j * 128 + r0, L)] for j in range(M)]
                t01 = (b[0] << 1) | b[1]
                t23 = (b[2] << 1) | b[3]
                t45 = (b[4] << 1) | b[5]
                t67 = (b[6] << 1) | b[7]
                idx = ((t01 << 6) | (t23 << 4)) | ((t45 << 2) | t67)
                pos = idx + ((idx >> 7) << 7)
                out_v[pl.ds(bout + r0, L)] = plsc.load_gather(tbl_v, [pos])
                out_v[pl.ds(bout + 128 + r0, L)] = plsc.load_gather(
                    tbl_v, [pos + 128]
                )

    pltpu.sync_copy(out_v, out_hbm.at[pl.ds(wid * (CHUNK * 2), CHUNK * 2)])


@jax.jit
def _mapper(bits_blocked, w_blocked):
    mesh = plsc.VectorSubcoreMesh(core_axis_name="c", subcore_axis_name="s")
    fn = pl.kernel(
        _sc_body,
        out_type=jax.ShapeDtypeStruct((2 * BATCH,), jnp.float32),
        mesh=mesh,
        scratch_types=[
            pltpu.VMEM((CHUNK * M,), jnp.int32),
            pltpu.VMEM((2 * NSYM,), jnp.float32),
            pltpu.VMEM((CHUNK * 2,), jnp.float32),
            pltpu.VMEM((128,), jnp.float32),
        ] + [pltpu.SemaphoreType.DMA] * NSUB,
        compiler_params=pltpu.CompilerParams(needs_layout_passes=False),
    )
    return fn(bits_blocked, w_blocked)


def kernel(b, weights):
    bits = b.reshape(BATCH // 128, 128, M).transpose(0, 2, 1).reshape(-1)
    w = weights.reshape(2, 128, 2).transpose(0, 2, 1).reshape(-1)
    out = _mapper(bits, w)
    return (
        out.reshape(BATCH // 128, 2, 128)
        .transpose(0, 2, 1)
        .reshape(BATCH, 1, 2)
    )


# speedup vs baseline: 2.6110x; 1.0017x over previous
import jax
import jax.numpy as jnp
from jax import lax
from jax.experimental import pallas as pl
from jax.experimental.pallas import tpu as pltpu
from jax.experimental.pallas import tpu_sc as plsc

M = 8
NSYM = 2 ** M
BATCH = 262144

NC, NS, L = 2, 16, 16
NW = NC * NS
CHUNK = BATCH // NW
BLOCKS = CHUNK // 128
BWORDS = 128 * M


def _rsqrt16(x):
    i = plsc.bitcast(x, jnp.int32)
    y = plsc.bitcast(jnp.int32(0x5F3759DF) - (i >> 1), jnp.float32)
    half = x * 0.5
    for _ in range(3):
        y = y * (1.5 - half * y * y)
    return y


def _allsum16(v, scratch):
    iota = lax.iota(jnp.int32, L)
    for k in (8, 4, 2, 1):
        scratch[pl.ds(0, L)] = v
        v = v + plsc.load_gather(scratch, [iota ^ k])
    return v


NSUB = 4
SBLK = BLOCKS // NSUB


def _sc_body(bits_hbm, w_hbm, out_hbm, bits_v, tbl_v, out_v, red_v, *sems):
    wid = lax.axis_index("s") * NC + lax.axis_index("c")
    ibase = wid * (CHUNK * M)

    copies = [
        pltpu.async_copy(
            bits_hbm.at[pl.ds(ibase + s * SBLK * BWORDS, SBLK * BWORDS)],
            bits_v.at[pl.ds(s * SBLK * BWORDS, SBLK * BWORDS)],
            sems[s],
        )
        for s in range(NSUB)
    ]
    pltpu.sync_copy(w_hbm, tbl_v)

    acc = jnp.zeros((L,), jnp.float32)
    for i in range(2 * NSYM // L):
        v = tbl_v[pl.ds(i * L, L)]
        acc = acc + v * v
    inv = _rsqrt16(_allsum16(acc, red_v) * (1.0 / NSYM))
    for i in range(2 * NSYM // L):
        tbl_v[pl.ds(i * L, L)] = tbl_v[pl.ds(i * L, L)] * inv

    for s in range(NSUB):
        copies[s].wait()

        @plsc.parallel_loop(s * SBLK, (s + 1) * SBLK)
        def _(t):
            bin_ = t * BWORDS
            bout = t * 256
            for gb in range(8):
                r0 = gb * L
                b = [bits_v[pl.ds(bin_ + j * 128 + r0, L)] for j in range(M)]
                t01 = (b[0] << 1) | b[1]
                t23 = (b[2] << 1) | b[3]
                t45 = (b[4] << 1) | b[5]
                t67 = (b[6] << 1) | b[7]
                idx = ((t01 << 6) | (t23 << 4)) | ((t45 << 2) | t67)
                pos = idx + ((idx >> 7) << 7)
                out_v[pl.ds(bout + r0, L)] = plsc.load_gather(tbl_v, [pos])
                out_v[pl.ds(bout + 128 + r0, L)] = plsc.load_gather(
                    tbl_v, [pos + 128]
                )

    pltpu.sync_copy(out_v, out_hbm.at[pl.ds(wid * (CHUNK * 2), CHUNK * 2)])


@jax.jit
def _mapper(bits_blocked, w_blocked):
    mesh = plsc.VectorSubcoreMesh(core_axis_name="c", subcore_axis_name="s")
    fn = pl.kernel(
        _sc_body,
        out_type=jax.ShapeDtypeStruct((2 * BATCH,), jnp.float32),
        mesh=mesh,
        scratch_types=[
            pltpu.VMEM((CHUNK * M,), jnp.int32),
            pltpu.VMEM((2 * NSYM,), jnp.float32),
            pltpu.VMEM((CHUNK * 2,), jnp.float32),
            pltpu.VMEM((128,), jnp.float32),
        ] + [pltpu.SemaphoreType.DMA] * NSUB,
        compiler_params=pltpu.CompilerParams(
            needs_layout_passes=False, skip_device_barrier=True
        ),
    )
    return fn(bits_blocked, w_blocked)


def kernel(b, weights):
    bits = b.reshape(BATCH // 128, 128, M).transpose(0, 2, 1).reshape(-1)
    w = weights.reshape(2, 128, 2).transpose(0, 2, 1).reshape(-1)
    out = _mapper(bits, w)
    return (
        out.reshape(BATCH // 128, 2, 128)
        .transpose(0, 2, 1)
        .reshape(BATCH, 1, 2)
    )


# speedup vs baseline: 2.8683x; 1.0986x over previous
import jax
import jax.numpy as jnp
from jax import lax
from jax.experimental import pallas as pl
from jax.experimental.pallas import tpu as pltpu
from jax.experimental.pallas import tpu_sc as plsc

M = 8
NSYM = 2 ** M
BATCH = 262144

NC, NS, L = 2, 16, 16
NW = NC * NS
CHUNK = BATCH // NW
BLOCKS = CHUNK // 128
BWORDS = 128 * M


def _rsqrt16(x):
    i = plsc.bitcast(x, jnp.int32)
    y = plsc.bitcast(jnp.int32(0x5F3759DF) - (i >> 1), jnp.float32)
    half = x * 0.5
    for _ in range(3):
        y = y * (1.5 - half * y * y)
    return y


def _allsum16(v, scratch):
    iota = lax.iota(jnp.int32, L)
    for k in (8, 4, 2, 1):
        scratch[pl.ds(0, L)] = v
        v = v + plsc.load_gather(scratch, [iota ^ k])
    return v


NSUB = 2
SBLK = BLOCKS // NSUB


def _sc_body(bits_hbm, w_hbm, out_hbm, bits_v, tbl_v, out_v, red_v, *sems):
    wid = lax.axis_index("s") * NC + lax.axis_index("c")
    ibase = wid * (CHUNK * M)

    copies = [
        pltpu.async_copy(
            bits_hbm.at[pl.ds(ibase + s * SBLK * BWORDS, SBLK * BWORDS)],
            bits_v.at[pl.ds(s * SBLK * BWORDS, SBLK * BWORDS)],
            sems[s],
        )
        for s in range(NSUB)
    ]
    pltpu.sync_copy(w_hbm, tbl_v)

    acc = jnp.zeros((L,), jnp.float32)
    for i in range(2 * NSYM // L):
        v = tbl_v[pl.ds(i * L, L)]
        acc = acc + v * v
    inv = _rsqrt16(_allsum16(acc, red_v) * (1.0 / NSYM))
    for i in range(2 * NSYM // L):
        tbl_v[pl.ds(i * L, L)] = tbl_v[pl.ds(i * L, L)] * inv

    obase = wid * (CHUNK * 2)
    out_copies = []
    for s in range(NSUB):
        copies[s].wait()

        @plsc.parallel_loop(s * SBLK, (s + 1) * SBLK)
        def _(t):
            bin_ = t * BWORDS
            bout = t * 256
            for gb in range(8):
                r0 = gb * L
                b = [bits_v[pl.ds(bin_ + j * 128 + r0, L)] for j in range(M)]
                t01 = (b[0] << 1) | b[1]
                t23 = (b[2] << 1) | b[3]
                t45 = (b[4] << 1) | b[5]
                t67 = (b[6] << 1) | b[7]
                idx = ((t01 << 6) | (t23 << 4)) | ((t45 << 2) | t67)
                pos = idx + ((idx >> 7) << 7)
                out_v[pl.ds(bout + r0, L)] = plsc.load_gather(tbl_v, [pos])
                out_v[pl.ds(bout + 128 + r0, L)] = plsc.load_gather(
                    tbl_v, [pos + 128]
                )

        out_copies.append(
            pltpu.async_copy(
                out_v.at[pl.ds(s * SBLK * 256, SBLK * 256)],
                out_hbm.at[pl.ds(obase + s * SBLK * 256, SBLK * 256)],
                sems[NSUB + s],
            )
        )
    for c in out_copies:
        c.wait()


@jax.jit
def _mapper(bits_blocked, w_blocked):
    mesh = plsc.VectorSubcoreMesh(core_axis_name="c", subcore_axis_name="s")
    fn = pl.kernel(
        _sc_body,
        out_type=jax.ShapeDtypeStruct((2 * BATCH,), jnp.float32),
        mesh=mesh,
        scratch_types=[
            pltpu.VMEM((CHUNK * M,), jnp.int32),
            pltpu.VMEM((2 * NSYM,), jnp.float32),
            pltpu.VMEM((CHUNK * 2,), jnp.float32),
            pltpu.VMEM((128,), jnp.float32),
        ] + [pltpu.SemaphoreType.DMA] * (2 * NSUB),
        compiler_params=pltpu.CompilerParams(needs_layout_passes=False),
    )
    return fn(bits_blocked, w_blocked)


def kernel(b, weights):
    bits = b.reshape(BATCH // 128, 128, M).transpose(0, 2, 1).reshape(-1)
    w = weights.reshape(2, 128, 2).transpose(0, 2, 1).reshape(-1)
    out = _mapper(bits, w)
    return (
        out.reshape(BATCH // 128, 2, 128)
        .transpose(0, 2, 1)
        .reshape(BATCH, 1, 2)
    )
